# Initial kernel scaffold; baseline (speedup 1.0000x reference)
import jax
import jax.numpy as jnp
from jax.experimental import pallas as pl


def kernel(vecs, codebook):
    raise NotImplementedError("write your pallas kernel here")


# speedup vs baseline: 2.4581x; 2.4581x over previous
import jax
import jax.numpy as jnp
from jax.experimental import pallas as pl

_B, _EMB = 1024, 768
_M, _K, _D = 96, 256, 8
_BB = 128


def _body(vt_ref, cb_ref, out_ref):
    v = vt_ref[...]
    cb = cb_ref[...]
    scores = jax.lax.dot_general(
        v, cb, (((2,), (2,)), ((0,), (0,))),
        preferred_element_type=jnp.float32,
        precision=jax.lax.Precision.HIGHEST)
    cn = jnp.sum(cb * cb, axis=-1)
    adj = 2.0 * scores - cn[:, None, :]
    amax = jnp.max(adj, axis=-1, keepdims=True)
    iota = jax.lax.broadcasted_iota(jnp.int32, adj.shape, 2)
    idx = jnp.min(jnp.where(adj == amax, iota, _K), axis=-1,
                  keepdims=True)
    oh = (iota == idx).astype(jnp.float32)
    out_ref[...] = jax.lax.dot_general(
        oh, cb, (((2,), (1,)), ((0,), (0,))),
        preferred_element_type=jnp.float32,
        precision=jax.lax.Precision.HIGHEST)


def kernel(vecs, codebook):
    vt = vecs.reshape(_B, _M, _D).transpose(1, 0, 2)
    q = pl.pallas_call(
        _body,
        grid=(_B // _BB,),
        in_specs=[
            pl.BlockSpec((_M, _BB, _D), lambda i: (0, i, 0)),
            pl.BlockSpec((_M, _K, _D), lambda i: (0, 0, 0)),
        ],
        out_specs=pl.BlockSpec((_M, _BB, _D), lambda i: (0, i, 0)),
        out_shape=jax.ShapeDtypeStruct((_M, _B, _D), jnp.float32),
    )(vt, codebook)
    return q.transpose(1, 0, 2).reshape(_B, _EMB)


# speedup vs baseline: 4.4354x; 1.8044x over previous
import jax
import jax.numpy as jnp
from jax.experimental import pallas as pl

_B, _EMB = 1024, 768
_M, _K, _D = 96, 256, 8
_BB = 128


def _body(vt_ref, cb_ref, cbt_ref, out_ref):
    v = vt_ref[...]
    cb = cb_ref[...]
    cbt = cbt_ref[...]
    scores = jax.lax.dot_general(
        cb, v, (((2,), (1,)), ((0,), (0,))),
        preferred_element_type=jnp.float32,
        precision=jax.lax.Precision.HIGHEST)
    cn = jnp.sum(cb * cb, axis=-1)
    adj = 2.0 * scores - cn[:, :, None]
    amax = jnp.max(adj, axis=1, keepdims=True)
    iota = jax.lax.broadcasted_iota(jnp.int32, adj.shape, 1)
    idx = jnp.min(jnp.where(adj == amax, iota, _K), axis=1,
                  keepdims=True)
    oh = (iota == idx).astype(jnp.float32)
    out_ref[...] = jax.lax.dot_general(
        cbt, oh, (((2,), (1,)), ((0,), (0,))),
        preferred_element_type=jnp.float32,
        precision=jax.lax.Precision.HIGHEST)


def kernel(vecs, codebook):
    vt = vecs.reshape(_B, _M, _D).transpose(1, 2, 0)
    cbt = codebook.transpose(0, 2, 1)
    q = pl.pallas_call(
        _body,
        grid=(_B // _BB,),
        in_specs=[
            pl.BlockSpec((_M, _D, _BB), lambda i: (0, 0, i)),
            pl.BlockSpec((_M, _K, _D), lambda i: (0, 0, 0)),
            pl.BlockSpec((_M, _D, _K), lambda i: (0, 0, 0)),
        ],
        out_specs=pl.BlockSpec((_M, _D, _BB), lambda i: (0, 0, i)),
        out_shape=jax.ShapeDtypeStruct((_M, _D, _B), jnp.float32),
    )(vt, codebook, cbt)
    return q.transpose(2, 0, 1).reshape(_B, _EMB)


# speedup vs baseline: 5.6345x; 1.2704x over previous
import jax
import jax.numpy as jnp
from jax.experimental import pallas as pl

_B, _EMB = 1024, 768
_M, _K, _D = 96, 256, 8
_BB = 128


def _body(vt_ref, cb_ref, cbt_ref, out_ref):
    v = vt_ref[...]
    cb = cb_ref[...]
    cbt = cbt_ref[...]
    scores = jax.lax.dot_general(
        cb, v, (((2,), (1,)), ((0,), (0,))),
        preferred_element_type=jnp.float32,
        precision=jax.lax.Precision.HIGHEST)
    cn = 0.5 * jnp.sum(cb * cb, axis=-1)
    adj = scores - cn[:, :, None]
    amax = jnp.max(adj, axis=1, keepdims=True)
    oh = (adj == amax).astype(jnp.float32)
    out_ref[...] = jax.lax.dot_general(
        cbt, oh, (((2,), (1,)), ((0,), (0,))),
        preferred_element_type=jnp.float32,
        precision=jax.lax.Precision.DEFAULT)


def kernel(vecs, codebook):
    vt = vecs.reshape(_B, _M, _D).transpose(1, 2, 0)
    cbt = codebook.transpose(0, 2, 1)
    q = pl.pallas_call(
        _body,
        grid=(_B // _BB,),
        in_specs=[
            pl.BlockSpec((_M, _D, _BB), lambda i: (0, 0, i)),
            pl.BlockSpec((_M, _K, _D), lambda i: (0, 0, 0)),
            pl.BlockSpec((_M, _D, _K), lambda i: (0, 0, 0)),
        ],
        out_specs=pl.BlockSpec((_M, _D, _BB), lambda i: (0, 0, i)),
        out_shape=jax.ShapeDtypeStruct((_M, _D, _B), jnp.float32),
    )(vt, codebook, cbt)
    return q.transpose(2, 0, 1).reshape(_B, _EMB)


# speedup vs baseline: 5.8064x; 1.0305x over previous
import jax
import jax.numpy as jnp
from jax.experimental import pallas as pl

_B, _EMB = 1024, 768
_M, _K, _D = 96, 256, 8
_BB = 128


def _body(vt_ref, cbx_ref, cbt_ref, out_ref):
    v = vt_ref[...]
    cbx = cbx_ref[...]
    cbt = cbt_ref[...]
    ones = jnp.ones((_M, _D, _BB), dtype=jnp.float32)
    vx = jnp.concatenate([v, ones], axis=1)
    adj = jax.lax.dot_general(
        cbx, vx, (((2,), (1,)), ((0,), (0,))),
        preferred_element_type=jnp.float32,
        precision=jax.lax.Precision.HIGHEST)
    amax = jnp.max(adj, axis=1, keepdims=True)
    oh = (adj == amax).astype(jnp.float32)
    out_ref[...] = jax.lax.dot_general(
        cbt, oh, (((2,), (1,)), ((0,), (0,))),
        preferred_element_type=jnp.float32,
        precision=jax.lax.Precision.DEFAULT)


def kernel(vecs, codebook):
    vt = vecs.reshape(_B, _M, _D).transpose(1, 2, 0)
    cbx = jnp.concatenate([codebook, -0.5 * codebook * codebook],
                          axis=2)
    cbt = codebook.transpose(0, 2, 1)
    q = pl.pallas_call(
        _body,
        grid=(_B // _BB,),
        in_specs=[
            pl.BlockSpec((_M, _D, _BB), lambda i: (0, 0, i)),
            pl.BlockSpec((_M, _K, 2 * _D), lambda i: (0, 0, 0)),
            pl.BlockSpec((_M, _D, _K), lambda i: (0, 0, 0)),
        ],
        out_specs=pl.BlockSpec((_M, _D, _BB), lambda i: (0, 0, i)),
        out_shape=jax.ShapeDtypeStruct((_M, _D, _B), jnp.float32),
    )(vt, cbx, cbt)
    return q.transpose(2, 0, 1).reshape(_B, _EMB)
